# baseline (speedup 1.0000x reference)
import jax
import jax.numpy as jnp
from jax.experimental import pallas as pl
from jax.experimental.pallas import tpu as pltpu

_CAP = 50000
_DIM = 1024
_B = 8
_BM0 = 2000
_BMN = 5000
_CHUNK = 1000
_AUG = _DIM + 3
_INV_EPS = 1e8
_LOG_EPS = -18.420680743952367


def _step0_body(cur_ref, mem_ref, out_ref, aug_ref, c_ref, m_ref, l_ref,
                acc_ref, cmax_ref):
    j = pl.program_id(0)
    nb = pl.num_programs(0)

    @pl.when(j == 0)
    def _init():
        m_ref[:] = jnp.full_like(m_ref, -jnp.inf)
        l_ref[:] = jnp.zeros_like(l_ref)
        acc_ref[:] = jnp.zeros_like(acc_ref)
        cmax_ref[:] = jnp.full_like(cmax_ref, -jnp.inf)

    cur = cur_ref[:]
    block = mem_ref[:]

    sq = block * block
    mn2_row = jax.lax.dot_general(
        jnp.ones((1, _DIM), jnp.float32), sq, (((1,), (1,)), ((), ())),
        preferred_element_type=jnp.float32)
    inv_mn_row = jnp.minimum(jax.lax.rsqrt(mn2_row), _INV_EPS)

    mn2_col = jnp.sum(sq, axis=1, keepdims=True)
    inv_col = jnp.minimum(jax.lax.rsqrt(mn2_col), _INV_EPS)
    logmn = jnp.maximum(0.5 * jnp.log(jnp.maximum(mn2_col, 0.0)), _LOG_EPS)
    hi = logmn.astype(jnp.bfloat16)
    lo = (logmn - hi.astype(jnp.float32)).astype(jnp.bfloat16)

    aug_ref[:, :_DIM] = (block * inv_col).astype(jnp.bfloat16)
    aug_ref[:, _DIM:_DIM + 1] = hi
    aug_ref[:, _DIM + 1:_DIM + 2] = lo
    aug_ref[:, _DIM + 2:_DIM + 3] = inv_col.astype(jnp.bfloat16)

    blk_max = jnp.max(0.5 * jnp.log(jnp.maximum(mn2_row, 1e-30)),
                      axis=1, keepdims=True)
    cmax_ref[:] = jnp.maximum(cmax_ref[:], blk_max)

    @pl.when(j == nb - 1)
    def _write_c():
        c_ref[:] = jnp.maximum(cmax_ref[:], _LOG_EPS) + 1.0

    cn2 = jnp.sum(cur * cur, axis=1, keepdims=True)
    inv_cn = jnp.minimum(jax.lax.rsqrt(cn2), _INV_EPS)
    dots = jax.lax.dot_general(
        cur, block, (((1,), (1,)), ((), ())),
        preferred_element_type=jnp.float32)
    sim = dots * inv_cn * inv_mn_row

    m_old = m_ref[:, :1]
    m_new = jnp.maximum(m_old, jnp.max(sim, axis=1, keepdims=True))
    p = jnp.exp(sim - m_new)
    corr = jnp.exp(m_old - m_new)
    l_new = l_ref[:, :1] * corr + jnp.sum(p, axis=1, keepdims=True)
    pv = jax.lax.dot_general(
        p, block, (((1,), (0,)), ((), ())),
        preferred_element_type=jnp.float32)
    acc_new = acc_ref[:] * corr + pv

    m_ref[:] = jnp.broadcast_to(m_new, m_ref.shape)
    l_ref[:] = jnp.broadcast_to(l_new, l_ref.shape)
    acc_ref[:] = acc_new

    @pl.when(j == nb - 1)
    def _finalize():
        attracted = acc_ref[:] / l_ref[:, :1]
        out_ref[:] = 0.8 * attracted + 0.2 * cur


def _attractor_blend(acc, cur):
    l = acc[:, _DIM + 2:_DIM + 3]
    attracted = acc[:, :_DIM] / l
    return 0.8 * attracted + 0.2 * cur


def _make_qaug(cur):
    cn2 = jnp.sum(cur * cur, axis=1, keepdims=True)
    inv_cn = jnp.minimum(jax.lax.rsqrt(cn2), _INV_EPS)
    ncur = cur * inv_cn
    return jnp.concatenate(
        [ncur, jnp.ones((_B, 2), jnp.float32), jnp.zeros((_B, 1), jnp.float32)],
        axis=1).astype(jnp.bfloat16)


def _qaug_pv(qaug, amat, c_shift):
    s = jax.lax.dot_general(
        qaug, amat, (((1,), (1,)), ((), ())),
        preferred_element_type=jnp.float32)

    p = jnp.exp(s - c_shift)
    return jax.lax.dot_general(
        p.astype(jnp.bfloat16), amat, (((1,), (0,)), ((), ())),
        preferred_element_type=jnp.float32)


def _stepn_body(cur_ref, aug_ref, c_ref, out_ref, acc_ref):
    j = pl.program_id(0)
    nb = pl.num_programs(0)

    @pl.when(j == 0)
    def _init():
        acc_ref[:] = jnp.zeros_like(acc_ref)

    cur = cur_ref[:]
    pv = _qaug_pv(_make_qaug(cur), aug_ref[:], c_ref[0:1, 0:1])
    acc_ref[:] = acc_ref[:] + pv

    @pl.when(j == nb - 1)
    def _finalize():
        out_ref[:] = _attractor_blend(acc_ref[:], cur)


def _step4_body(cur_in_ref, auga_ref, augb_ref, c_ref, out_ref, acc_ref,
                cur_ref):
    t = pl.program_id(0)
    nt = pl.num_programs(0)
    j = pl.program_id(1)
    nb = pl.num_programs(1)

    @pl.when(j == 0)
    def _newstep():
        @pl.when(t == 0)
        def _load():
            cur_ref[:] = cur_in_ref[:]

        @pl.when(t > 0)
        def _advance():
            cur_ref[:] = _attractor_blend(acc_ref[:], cur_ref[:])

        acc_ref[:] = jnp.zeros_like(acc_ref)

    cur = cur_ref[:]
    qaug = _make_qaug(cur)
    c_shift = c_ref[0:1, 0:1]
    pv = (_qaug_pv(qaug, auga_ref[:], c_shift) +
          _qaug_pv(qaug, augb_ref[:], c_shift))
    acc_ref[:] = acc_ref[:] + pv

    @pl.when((t == nt - 1) & (j == nb - 1))
    def _finalize():
        out_ref[:] = _attractor_blend(acc_ref[:], cur)


def _step0(cur, memory_matrix):
    nb = _CAP // _BM0
    return pl.pallas_call(
        _step0_body,
        grid=(nb,),
        in_specs=[
            pl.BlockSpec((_B, _DIM), lambda j: (0, 0)),
            pl.BlockSpec((_BM0, _DIM), lambda j: (j, 0)),
        ],
        out_specs=[
            pl.BlockSpec((_B, _DIM), lambda j: (0, 0)),
            pl.BlockSpec((_BM0, _AUG), lambda j: (j, 0)),
            pl.BlockSpec((1, 128), lambda j: (0, 0)),
        ],
        out_shape=[
            jax.ShapeDtypeStruct((_B, _DIM), jnp.float32),
            jax.ShapeDtypeStruct((_CAP, _AUG), jnp.bfloat16),
            jax.ShapeDtypeStruct((1, 128), jnp.float32),
        ],
        scratch_shapes=[
            pltpu.VMEM((_B, 128), jnp.float32),
            pltpu.VMEM((_B, 128), jnp.float32),
            pltpu.VMEM((_B, _DIM), jnp.float32),
            pltpu.VMEM((1, 128), jnp.float32),
        ],
        compiler_params=pltpu.CompilerParams(
            dimension_semantics=("arbitrary",),
        ),
    )(cur, memory_matrix)


def _stepn(cur, aug_matrix, c_shift):
    nb = _CAP // _BMN
    return pl.pallas_call(
        _stepn_body,
        grid=(nb,),
        in_specs=[
            pl.BlockSpec((_B, _DIM), lambda j: (0, 0)),
            pl.BlockSpec((_BMN, _AUG), lambda j: (j, 0)),
            pl.BlockSpec((1, 128), lambda j: (0, 0)),
        ],
        out_specs=pl.BlockSpec((_B, _DIM), lambda j: (0, 0)),
        out_shape=jax.ShapeDtypeStruct((_B, _DIM), jnp.float32),
        scratch_shapes=[
            pltpu.VMEM((_B, _AUG), jnp.float32),
        ],
        compiler_params=pltpu.CompilerParams(
            dimension_semantics=("arbitrary",),
        ),
    )(cur, aug_matrix, c_shift)


_FUSE = 4


def _step4(cur, aug_matrix, c_shift):
    nb = _CAP // (2 * _BMN)
    return pl.pallas_call(
        _step4_body,
        grid=(_FUSE, nb),
        in_specs=[
            pl.BlockSpec((_B, _DIM), lambda t, j: (0, 0)),
            pl.BlockSpec((_BMN, _AUG), lambda t, j: (2 * j, 0)),
            pl.BlockSpec((_BMN, _AUG), lambda t, j: (2 * j + 1, 0)),
            pl.BlockSpec((1, 128), lambda t, j: (0, 0)),
        ],
        out_specs=pl.BlockSpec((_B, _DIM), lambda t, j: (0, 0)),
        out_shape=jax.ShapeDtypeStruct((_B, _DIM), jnp.float32),
        scratch_shapes=[
            pltpu.VMEM((_B, _AUG), jnp.float32),
            pltpu.VMEM((_B, _DIM), jnp.float32),
        ],
        compiler_params=pltpu.CompilerParams(
            dimension_semantics=("arbitrary", "arbitrary"),
        ),
    )(cur, aug_matrix, aug_matrix, c_shift)


def kernel(query_trace, memory_matrix, steps):
    cur1, aug_matrix, c_shift = _step0(query_trace, memory_matrix)

    inner = steps - 1
    nfull = jnp.maximum(inner, 0) // _FUSE
    nrem = jnp.maximum(inner, 0) % _FUSE

    cur = jax.lax.fori_loop(
        0, nfull, lambda _, c: _step4(c, aug_matrix, c_shift), cur1)
    cur = jax.lax.fori_loop(
        0, nrem, lambda _, c: _stepn(c, aug_matrix, c_shift), cur)
    return jnp.where(steps >= 1, cur, query_trace)


# speedup vs baseline: 1.0120x; 1.0120x over previous
import jax
import jax.numpy as jnp
from jax.experimental import pallas as pl
from jax.experimental.pallas import tpu as pltpu

_CAP = 50000
_DIM = 1024
_B = 8
_BM0 = 2000
_BMN = 5000
_CHUNK = 1000
_AUG = _DIM + 3
_INV_EPS = 1e8
_LOG_EPS = -18.420680743952367


def _step0_body(cur_ref, mem_ref, out_ref, aug_ref, c_ref, m_ref, l_ref,
                acc_ref, cmax_ref):
    j = pl.program_id(0)
    nb = pl.num_programs(0)

    @pl.when(j == 0)
    def _init():
        m_ref[:] = jnp.full_like(m_ref, -jnp.inf)
        l_ref[:] = jnp.zeros_like(l_ref)
        acc_ref[:] = jnp.zeros_like(acc_ref)
        cmax_ref[:] = jnp.full_like(cmax_ref, -jnp.inf)

    cur = cur_ref[:]
    block = mem_ref[:]

    sq = block * block
    mn2_row = jax.lax.dot_general(
        jnp.ones((1, _DIM), jnp.float32), sq, (((1,), (1,)), ((), ())),
        preferred_element_type=jnp.float32)
    inv_mn_row = jnp.minimum(jax.lax.rsqrt(mn2_row), _INV_EPS)

    mn2_col = jnp.sum(sq, axis=1, keepdims=True)
    inv_col = jnp.minimum(jax.lax.rsqrt(mn2_col), _INV_EPS)
    logmn = jnp.maximum(0.5 * jnp.log(jnp.maximum(mn2_col, 0.0)), _LOG_EPS)
    hi = logmn.astype(jnp.bfloat16)
    lo = (logmn - hi.astype(jnp.float32)).astype(jnp.bfloat16)

    aug_ref[:, :_DIM] = (block * inv_col).astype(jnp.bfloat16)
    aug_ref[:, _DIM:_DIM + 1] = hi
    aug_ref[:, _DIM + 1:_DIM + 2] = lo
    aug_ref[:, _DIM + 2:_DIM + 3] = inv_col.astype(jnp.bfloat16)

    blk_max = jnp.max(0.5 * jnp.log(jnp.maximum(mn2_row, 1e-30)),
                      axis=1, keepdims=True)
    cmax_ref[:] = jnp.maximum(cmax_ref[:], blk_max)

    @pl.when(j == nb - 1)
    def _write_c():
        c_ref[:] = jnp.maximum(cmax_ref[:], _LOG_EPS) + 1.0

    cn2 = jnp.sum(cur * cur, axis=1, keepdims=True)
    inv_cn = jnp.minimum(jax.lax.rsqrt(cn2), _INV_EPS)
    dots = jax.lax.dot_general(
        cur, block, (((1,), (1,)), ((), ())),
        preferred_element_type=jnp.float32)
    sim = dots * inv_cn * inv_mn_row

    m_old = m_ref[:, :1]
    m_new = jnp.maximum(m_old, jnp.max(sim, axis=1, keepdims=True))
    p = jnp.exp(sim - m_new)
    corr = jnp.exp(m_old - m_new)
    l_new = l_ref[:, :1] * corr + jnp.sum(p, axis=1, keepdims=True)
    pv = jax.lax.dot_general(
        p, block, (((1,), (0,)), ((), ())),
        preferred_element_type=jnp.float32)
    acc_new = acc_ref[:] * corr + pv

    m_ref[:] = jnp.broadcast_to(m_new, m_ref.shape)
    l_ref[:] = jnp.broadcast_to(l_new, l_ref.shape)
    acc_ref[:] = acc_new

    @pl.when(j == nb - 1)
    def _finalize():
        attracted = acc_ref[:] / l_ref[:, :1]
        out_ref[:] = 0.8 * attracted + 0.2 * cur


def _attractor_blend(acc, cur):
    l = acc[:, _DIM + 2:_DIM + 3]
    attracted = acc[:, :_DIM] / l
    return 0.8 * attracted + 0.2 * cur


def _make_qaug(cur):
    cn2 = jnp.sum(cur * cur, axis=1, keepdims=True)
    inv_cn = jnp.minimum(jax.lax.rsqrt(cn2), _INV_EPS)
    ncur = cur * inv_cn
    return jnp.concatenate(
        [ncur, jnp.ones((_B, 2), jnp.float32), jnp.zeros((_B, 1), jnp.float32)],
        axis=1).astype(jnp.bfloat16)


def _qaug_pv(qaug, amat, c_shift):
    s = jax.lax.dot_general(
        qaug, amat, (((1,), (1,)), ((), ())),
        preferred_element_type=jnp.float32)

    p = jnp.exp(s - c_shift)
    return jax.lax.dot_general(
        p.astype(jnp.bfloat16), amat, (((1,), (0,)), ((), ())),
        preferred_element_type=jnp.float32)


def _stepn_body(cur_ref, aug_ref, c_ref, out_ref, acc_ref):
    j = pl.program_id(0)
    nb = pl.num_programs(0)

    @pl.when(j == 0)
    def _init():
        acc_ref[:] = jnp.zeros_like(acc_ref)

    cur = cur_ref[:]
    pv = _qaug_pv(_make_qaug(cur), aug_ref[:], c_ref[0:1, 0:1])
    acc_ref[:] = acc_ref[:] + pv

    @pl.when(j == nb - 1)
    def _finalize():
        out_ref[:] = _attractor_blend(acc_ref[:], cur)


def _step4_body(cur_in_ref, aug_ref, c_ref, out_ref, acc_ref, cur_ref):
    t = pl.program_id(0)
    nt = pl.num_programs(0)
    j = pl.program_id(1)
    nb = pl.num_programs(1)

    @pl.when(j == 0)
    def _newstep():
        @pl.when(t == 0)
        def _load():
            cur_ref[:] = cur_in_ref[:]

        @pl.when(t > 0)
        def _advance():
            cur_ref[:] = _attractor_blend(acc_ref[:], cur_ref[:])

        acc_ref[:] = jnp.zeros_like(acc_ref)

    cur = cur_ref[:]
    pv = _qaug_pv(_make_qaug(cur), aug_ref[:], c_ref[0:1, 0:1])
    acc_ref[:] = acc_ref[:] + pv

    @pl.when((t == nt - 1) & (j == nb - 1))
    def _finalize():
        out_ref[:] = _attractor_blend(acc_ref[:], cur)


def _step0(cur, memory_matrix):
    nb = _CAP // _BM0
    return pl.pallas_call(
        _step0_body,
        grid=(nb,),
        in_specs=[
            pl.BlockSpec((_B, _DIM), lambda j: (0, 0)),
            pl.BlockSpec((_BM0, _DIM), lambda j: (j, 0)),
        ],
        out_specs=[
            pl.BlockSpec((_B, _DIM), lambda j: (0, 0)),
            pl.BlockSpec((_BM0, _AUG), lambda j: (j, 0)),
            pl.BlockSpec((1, 128), lambda j: (0, 0)),
        ],
        out_shape=[
            jax.ShapeDtypeStruct((_B, _DIM), jnp.float32),
            jax.ShapeDtypeStruct((_CAP, _AUG), jnp.bfloat16),
            jax.ShapeDtypeStruct((1, 128), jnp.float32),
        ],
        scratch_shapes=[
            pltpu.VMEM((_B, 128), jnp.float32),
            pltpu.VMEM((_B, 128), jnp.float32),
            pltpu.VMEM((_B, _DIM), jnp.float32),
            pltpu.VMEM((1, 128), jnp.float32),
        ],
        compiler_params=pltpu.CompilerParams(
            dimension_semantics=("arbitrary",),
        ),
    )(cur, memory_matrix)


def _stepn(cur, aug_matrix, c_shift):
    nb = _CAP // _BMN
    return pl.pallas_call(
        _stepn_body,
        grid=(nb,),
        in_specs=[
            pl.BlockSpec((_B, _DIM), lambda j: (0, 0)),
            pl.BlockSpec((_BMN, _AUG), lambda j: (j, 0)),
            pl.BlockSpec((1, 128), lambda j: (0, 0)),
        ],
        out_specs=pl.BlockSpec((_B, _DIM), lambda j: (0, 0)),
        out_shape=jax.ShapeDtypeStruct((_B, _DIM), jnp.float32),
        scratch_shapes=[
            pltpu.VMEM((_B, _AUG), jnp.float32),
        ],
        compiler_params=pltpu.CompilerParams(
            dimension_semantics=("arbitrary",),
        ),
    )(cur, aug_matrix, c_shift)


_FUSE = 4


def _step4(cur, aug_matrix, c_shift):
    nb = _CAP // _BMN
    return pl.pallas_call(
        _step4_body,
        grid=(_FUSE, nb),
        in_specs=[
            pl.BlockSpec((_B, _DIM), lambda t, j: (0, 0)),
            pl.BlockSpec((_BMN, _AUG), lambda t, j: (j, 0)),
            pl.BlockSpec((1, 128), lambda t, j: (0, 0)),
        ],
        out_specs=pl.BlockSpec((_B, _DIM), lambda t, j: (0, 0)),
        out_shape=jax.ShapeDtypeStruct((_B, _DIM), jnp.float32),
        scratch_shapes=[
            pltpu.VMEM((_B, _AUG), jnp.float32),
            pltpu.VMEM((_B, _DIM), jnp.float32),
        ],
        compiler_params=pltpu.CompilerParams(
            dimension_semantics=("arbitrary", "arbitrary"),
        ),
    )(cur, aug_matrix, c_shift)


def kernel(query_trace, memory_matrix, steps):
    cur1, aug_matrix, c_shift = _step0(query_trace, memory_matrix)

    inner = steps - 1
    nfull = jnp.maximum(inner, 0) // _FUSE
    nrem = jnp.maximum(inner, 0) % _FUSE

    cur = jax.lax.fori_loop(
        0, nfull, lambda _, c: _step4(c, aug_matrix, c_shift), cur1)
    cur = jax.lax.fori_loop(
        0, nrem, lambda _, c: _stepn(c, aug_matrix, c_shift), cur)
    return jnp.where(steps >= 1, cur, query_trace)


# speedup vs baseline: 1.1081x; 1.0950x over previous
import jax
import jax.numpy as jnp
from jax.experimental import pallas as pl
from jax.experimental.pallas import tpu as pltpu

_CAP = 50000
_DIM = 1024
_B = 8
_BM0 = 2000
_BMN = 5000
_SC = 1000
_INV_EPS = 1e8
_LOG_EPS = -18.420680743952367


def _step0_body(cur_ref, mem_ref, out_ref, nmat_ref, logside_ref, invside_ref,
                c_ref, m_ref, l_ref, acc_ref, cmax_ref):
    j = pl.program_id(0)
    nb = pl.num_programs(0)

    @pl.when(j == 0)
    def _init():
        m_ref[:] = jnp.full_like(m_ref, -jnp.inf)
        l_ref[:] = jnp.zeros_like(l_ref)
        acc_ref[:] = jnp.zeros_like(acc_ref)
        cmax_ref[:] = jnp.full_like(cmax_ref, -jnp.inf)

    cur = cur_ref[:]
    block = mem_ref[:]

    sq = block * block
    mn2_row = jax.lax.dot_general(
        jnp.ones((1, _DIM), jnp.float32), sq, (((1,), (1,)), ((), ())),
        preferred_element_type=jnp.float32)
    inv_mn_row = jnp.minimum(jax.lax.rsqrt(mn2_row), _INV_EPS)
    logmn_row = jnp.maximum(0.5 * jnp.log(jnp.maximum(mn2_row, 1e-30)),
                            _LOG_EPS)

    mn2_col = jnp.sum(sq, axis=1, keepdims=True)
    inv_col = jnp.minimum(jax.lax.rsqrt(mn2_col), _INV_EPS)

    nmat_ref[:] = (block * inv_col).astype(jnp.bfloat16)
    for a in range(_BM0 // _SC):
        sl = slice(a * _SC, (a + 1) * _SC)
        logside_ref[a, :, :] = logmn_row[:, sl]
        invside_ref[a, :, :] = inv_mn_row[:, sl]

    cmax_ref[:] = jnp.maximum(cmax_ref[:],
                              jnp.max(logmn_row, axis=1, keepdims=True))

    @pl.when(j == nb - 1)
    def _write_c():
        c_ref[:] = cmax_ref[:] + 1.0

    cn2 = jnp.sum(cur * cur, axis=1, keepdims=True)
    inv_cn = jnp.minimum(jax.lax.rsqrt(cn2), _INV_EPS)
    dots = jax.lax.dot_general(
        cur, block, (((1,), (1,)), ((), ())),
        preferred_element_type=jnp.float32)
    sim = dots * inv_cn * inv_mn_row

    m_old = m_ref[:, :1]
    m_new = jnp.maximum(m_old, jnp.max(sim, axis=1, keepdims=True))
    p = jnp.exp(sim - m_new)
    corr = jnp.exp(m_old - m_new)
    l_new = l_ref[:, :1] * corr + jnp.sum(p, axis=1, keepdims=True)
    pv = jax.lax.dot_general(
        p, block, (((1,), (0,)), ((), ())),
        preferred_element_type=jnp.float32)
    acc_new = acc_ref[:] * corr + pv

    m_ref[:] = jnp.broadcast_to(m_new, m_ref.shape)
    l_ref[:] = jnp.broadcast_to(l_new, l_ref.shape)
    acc_ref[:] = acc_new

    @pl.when(j == nb - 1)
    def _finalize():
        attracted = acc_ref[:] / l_ref[:, :1]
        out_ref[:] = 0.8 * attracted + 0.2 * cur


def _ncur_bf16(cur):
    cn2 = jnp.sum(cur * cur, axis=1, keepdims=True)
    inv_cn = jnp.minimum(jax.lax.rsqrt(cn2), _INV_EPS)
    return (cur * inv_cn).astype(jnp.bfloat16)


def _step_block(cur, nmat, logside, invside, c_shift, acc_ref, lacc_ref):
    logrow = jnp.concatenate(
        [logside[a, :, :] for a in range(_BMN // _SC)], axis=1)
    invrow = jnp.concatenate(
        [invside[a, :, :] for a in range(_BMN // _SC)], axis=1)

    s = jax.lax.dot_general(
        _ncur_bf16(cur), nmat, (((1,), (1,)), ((), ())),
        preferred_element_type=jnp.float32)
    p = jnp.exp(s + logrow - c_shift)
    pv = jax.lax.dot_general(
        p.astype(jnp.bfloat16), nmat, (((1,), (0,)), ((), ())),
        preferred_element_type=jnp.float32)
    acc_ref[:] = acc_ref[:] + pv
    l_part = jnp.sum(p * invrow, axis=1, keepdims=True)
    lacc_ref[:] = lacc_ref[:] + jnp.broadcast_to(l_part, lacc_ref.shape)


def _blend(acc_ref, lacc_ref, cur):
    attracted = acc_ref[:] / lacc_ref[:, :1]
    return 0.8 * attracted + 0.2 * cur


def _stepn_body(cur_ref, nmat_ref, logside_ref, invside_ref, c_ref, out_ref,
                acc_ref, lacc_ref):
    j = pl.program_id(0)
    nb = pl.num_programs(0)

    @pl.when(j == 0)
    def _init():
        acc_ref[:] = jnp.zeros_like(acc_ref)
        lacc_ref[:] = jnp.zeros_like(lacc_ref)

    cur = cur_ref[:]
    _step_block(cur, nmat_ref[:], logside_ref, invside_ref, c_ref[0:1, 0:1],
                acc_ref, lacc_ref)

    @pl.when(j == nb - 1)
    def _finalize():
        out_ref[:] = _blend(acc_ref, lacc_ref, cur)


def _step4_body(cur_in_ref, nmat_ref, logside_ref, invside_ref, c_ref,
                out_ref, acc_ref, lacc_ref, cur_ref):
    t = pl.program_id(0)
    nt = pl.num_programs(0)
    j = pl.program_id(1)
    nb = pl.num_programs(1)

    @pl.when(j == 0)
    def _newstep():
        @pl.when(t == 0)
        def _load():
            cur_ref[:] = cur_in_ref[:]

        @pl.when(t > 0)
        def _advance():
            cur_ref[:] = _blend(acc_ref, lacc_ref, cur_ref[:])

        acc_ref[:] = jnp.zeros_like(acc_ref)
        lacc_ref[:] = jnp.zeros_like(lacc_ref)

    cur = cur_ref[:]
    _step_block(cur, nmat_ref[:], logside_ref, invside_ref, c_ref[0:1, 0:1],
                acc_ref, lacc_ref)

    @pl.when((t == nt - 1) & (j == nb - 1))
    def _finalize():
        out_ref[:] = _blend(acc_ref, lacc_ref, cur)


def _step0(cur, memory_matrix):
    nb = _CAP // _BM0
    nsc = _CAP // _SC
    return pl.pallas_call(
        _step0_body,
        grid=(nb,),
        in_specs=[
            pl.BlockSpec((_B, _DIM), lambda j: (0, 0)),
            pl.BlockSpec((_BM0, _DIM), lambda j: (j, 0)),
        ],
        out_specs=[
            pl.BlockSpec((_B, _DIM), lambda j: (0, 0)),
            pl.BlockSpec((_BM0, _DIM), lambda j: (j, 0)),
            pl.BlockSpec((_BM0 // _SC, 1, _SC), lambda j: (j, 0, 0)),
            pl.BlockSpec((_BM0 // _SC, 1, _SC), lambda j: (j, 0, 0)),
            pl.BlockSpec((1, 128), lambda j: (0, 0)),
        ],
        out_shape=[
            jax.ShapeDtypeStruct((_B, _DIM), jnp.float32),
            jax.ShapeDtypeStruct((_CAP, _DIM), jnp.bfloat16),
            jax.ShapeDtypeStruct((nsc, 1, _SC), jnp.float32),
            jax.ShapeDtypeStruct((nsc, 1, _SC), jnp.float32),
            jax.ShapeDtypeStruct((1, 128), jnp.float32),
        ],
        scratch_shapes=[
            pltpu.VMEM((_B, 128), jnp.float32),
            pltpu.VMEM((_B, 128), jnp.float32),
            pltpu.VMEM((_B, _DIM), jnp.float32),
            pltpu.VMEM((1, 128), jnp.float32),
        ],
        compiler_params=pltpu.CompilerParams(
            dimension_semantics=("arbitrary",),
        ),
    )(cur, memory_matrix)


_FUSE = 4


def _stepn(cur, nmat, logside, invside, c_shift):
    nb = _CAP // _BMN
    return pl.pallas_call(
        _stepn_body,
        grid=(nb,),
        in_specs=[
            pl.BlockSpec((_B, _DIM), lambda j: (0, 0)),
            pl.BlockSpec((_BMN, _DIM), lambda j: (j, 0)),
            pl.BlockSpec((_BMN // _SC, 1, _SC), lambda j: (j, 0, 0)),
            pl.BlockSpec((_BMN // _SC, 1, _SC), lambda j: (j, 0, 0)),
            pl.BlockSpec((1, 128), lambda j: (0, 0)),
        ],
        out_specs=pl.BlockSpec((_B, _DIM), lambda j: (0, 0)),
        out_shape=jax.ShapeDtypeStruct((_B, _DIM), jnp.float32),
        scratch_shapes=[
            pltpu.VMEM((_B, _DIM), jnp.float32),
            pltpu.VMEM((_B, 128), jnp.float32),
        ],
        compiler_params=pltpu.CompilerParams(
            dimension_semantics=("arbitrary",),
        ),
    )(cur, nmat, logside, invside, c_shift)


def _step4(cur, nmat, logside, invside, c_shift):
    nb = _CAP // _BMN
    return pl.pallas_call(
        _step4_body,
        grid=(_FUSE, nb),
        in_specs=[
            pl.BlockSpec((_B, _DIM), lambda t, j: (0, 0)),
            pl.BlockSpec((_BMN, _DIM), lambda t, j: (j, 0)),
            pl.BlockSpec((_BMN // _SC, 1, _SC), lambda t, j: (j, 0, 0)),
            pl.BlockSpec((_BMN // _SC, 1, _SC), lambda t, j: (j, 0, 0)),
            pl.BlockSpec((1, 128), lambda t, j: (0, 0)),
        ],
        out_specs=pl.BlockSpec((_B, _DIM), lambda t, j: (0, 0)),
        out_shape=jax.ShapeDtypeStruct((_B, _DIM), jnp.float32),
        scratch_shapes=[
            pltpu.VMEM((_B, _DIM), jnp.float32),
            pltpu.VMEM((_B, 128), jnp.float32),
            pltpu.VMEM((_B, _DIM), jnp.float32),
        ],
        compiler_params=pltpu.CompilerParams(
            dimension_semantics=("arbitrary", "arbitrary"),
        ),
    )(cur, nmat, logside, invside, c_shift)


def kernel(query_trace, memory_matrix, steps):
    cur1, nmat, logside, invside, c_shift = _step0(query_trace, memory_matrix)

    inner = steps - 1
    nfull = jnp.maximum(inner, 0) // _FUSE
    nrem = jnp.maximum(inner, 0) % _FUSE

    cur = jax.lax.fori_loop(
        0, nfull, lambda _, c: _step4(c, nmat, logside, invside, c_shift),
        cur1)
    cur = jax.lax.fori_loop(
        0, nrem, lambda _, c: _stepn(c, nmat, logside, invside, c_shift),
        cur)
    return jnp.where(steps >= 1, cur, query_trace)
